# baseline (speedup 1.0000x reference)
import functools

import jax
import jax.numpy as jnp
from jax import lax
from jax.experimental import pallas as pl
from jax.experimental.pallas import tpu as pltpu
from jax.experimental.pallas import tpu_sc as plsc

_EMB = 64
_SCALE = 8.0

_NC = 2
_NS = 16
_NW = _NC * _NS

_SEQ = 200
_BATCH = 16384
_CHUNK = 256
_LANE = 128
_TE = _EMB // 8
_QROW = _CHUNK // 8


@functools.lru_cache(maxsize=None)
def _emb_kernel():
    bpw = _BATCH // _NW
    cps = bpw // _CHUNK
    nch = _SEQ * cps
    nq = _CHUNK // _LANE * 8
    mesh = plsc.VectorSubcoreMesh(core_axis_name="c", subcore_axis_name="s")

    @functools.partial(
        pl.kernel,
        mesh=mesh,
        compiler_params=pltpu.CompilerParams(use_tc_tiling_on_sc=False,
                                             needs_layout_passes=False),
        out_type=jax.ShapeDtypeStruct((_SEQ, _TE, 1024, _LANE), jnp.float32),
        scratch_types=[
            pltpu.VMEM((4, _CHUNK), jnp.int32),
            pltpu.VMEM((4, _CHUNK, _EMB), jnp.float32),
            pltpu.VMEM((2, _TE, nq, _LANE), jnp.float32),
            pltpu.SemaphoreType.DMA((4,)),
            pltpu.SemaphoreType.DMA((2,)),
        ],
    )
    def k(tok_hbm, table_hbm, out_hbm, idx_v, rows_v, t_v, gsem, wsem):
        wid = lax.axis_index("s") * _NC + lax.axis_index("c")

        def tok_off(ck):
            s = ck // cps
            c = ck % cps
            return s * _BATCH + wid * bpw + c * _CHUNK

        def start_gather(ck, b):
            off = tok_off(ck)
            pltpu.sync_copy(tok_hbm.at[pl.ds(off, _CHUNK)], idx_v.at[b])
            pltpu.async_copy(table_hbm.at[idx_v.at[b]], rows_v.at[b],
                             gsem.at[b])

        def out_slice(ck):
            s = ck // cps
            c = ck % cps
            q0 = wid * (bpw // _LANE * 8) + c * nq
            return out_hbm.at[s, :, pl.ds(q0, nq), :]

        for b in range(3):
            start_gather(b, b)

        def body(g, _):
            for b in range(4):
                ck = g * 4 + b
                tb = b % 2

                @pl.when(ck < nch - 3)
                def _():
                    start_gather(ck + 3, (b + 3) % 4)

                pltpu.make_async_copy(
                    table_hbm.at[idx_v.at[b]], rows_v.at[b],
                    gsem.at[b]).wait()

                @pl.when(ck >= 2)
                def _():
                    pltpu.make_async_copy(
                        t_v.at[tb], out_slice(ck - 2), wsem.at[tb]).wait()

                @plsc.parallel_loop(0, _CHUNK * 4, 1, unroll=8)
                def _(j):
                    te = j >> 7
                    tcp = (j >> 6) & (_CHUNK // _LANE - 1)
                    ep = (j >> 3) & 7
                    rb = j & 7
                    ridx = tcp * _LANE + rb * 16 + lax.iota(jnp.int32, 16)
                    cidx = jnp.full((16,), te * 8 + ep, jnp.int32)
                    vals = plsc.load_gather(rows_v.at[b], [ridx, cidx])
                    t_v[tb, te, tcp * 8 + ep, pl.ds(rb * 16, 16)] = (
                        vals * _SCALE)

                pltpu.async_copy(t_v.at[tb], out_slice(ck), wsem.at[tb])
            return 0

        lax.fori_loop(0, nch // 4, body, 0)

        for ck in (nch - 2, nch - 1):
            pltpu.make_async_copy(
                t_v.at[ck % 2], out_slice(ck), wsem.at[ck % 2]).wait()

    return k


@jax.jit
def kernel(tokens, table):
    tok = jnp.transpose(tokens, (1, 0)).reshape(-1).astype(jnp.int32)
    x = _emb_kernel()(tok, table)
    x5 = x.reshape(_SEQ, _TE, _LANE, 8, _LANE)
    return jnp.transpose(x5, (2, 4, 0, 1, 3)).reshape(_BATCH, _SEQ, _EMB)


# speedup vs baseline: 1.3967x; 1.3967x over previous
import functools

import jax
import jax.numpy as jnp
from jax import lax
from jax.experimental import pallas as pl
from jax.experimental.pallas import tpu as pltpu
from jax.experimental.pallas import tpu_sc as plsc

_EMB = 64
_SCALE = 8.0

_NC = 2
_NS = 16
_NW = _NC * _NS

_SEQ = 200
_BATCH = 16384
_B = _SEQ * _BATCH
_CHUNK = 256


@functools.lru_cache(maxsize=None)
def _gather_kernel():
    bpw = _B // _NW
    nch = bpw // _CHUNK
    half = _CHUNK // 2
    mesh = plsc.VectorSubcoreMesh(core_axis_name="c", subcore_axis_name="s")

    @functools.partial(
        pl.kernel,
        mesh=mesh,
        compiler_params=pltpu.CompilerParams(use_tc_tiling_on_sc=False,
                                             needs_layout_passes=False),
        out_type=jax.ShapeDtypeStruct((_B, _EMB), jnp.float32),
        scratch_types=[
            pltpu.VMEM((4, 2, half), jnp.int32),
            pltpu.VMEM((4, _CHUNK), jnp.int32),
            pltpu.VMEM((4, _CHUNK, _EMB), jnp.float32),
            pltpu.SemaphoreType.DMA((4,)),
            pltpu.SemaphoreType.DMA((4,)),
        ],
    )
    def k(tok_hbm, table_hbm, out_hbm, stage_v, idx_v, rows_v, gsem, ssem):
        wid = lax.axis_index("s") * _NC + lax.axis_index("c")
        base = wid * bpw

        def start_gather(ck, b):
            j0 = (base + ck * _CHUNK) // 2
            s = j0 >> 13
            rem = j0 & 8191
            blk = rem >> 9
            a_off = pl.multiple_of(s * _BATCH + blk * 1024 + (rem & 511), 128)
            pltpu.sync_copy(tok_hbm.at[pl.ds(a_off, half)],
                            stage_v.at[b, 0])
            pltpu.sync_copy(tok_hbm.at[pl.ds(pl.multiple_of(a_off + 512, 128),
                                             half)],
                            stage_v.at[b, 1])
            for gi in range(half // 16):
                lanes = gi * 16 + lax.iota(jnp.int32, 16)
                va = stage_v[b, 0, pl.ds(gi * 16, 16)]
                vb = stage_v[b, 1, pl.ds(gi * 16, 16)]
                plsc.store_scatter(idx_v.at[b], [lanes * 2], va)
                plsc.store_scatter(idx_v.at[b], [lanes * 2 + 1], vb)
            pltpu.async_copy(table_hbm.at[idx_v.at[b]], rows_v.at[b],
                             gsem.at[b])

        for b in range(2):
            start_gather(b, b)

        def body(g, _):
            for b in range(4):
                ck = g * 4 + b
                b2 = (b + 2) % 4

                @pl.when(ck >= 2)
                def _():
                    off2 = base + (ck - 2) * _CHUNK
                    pltpu.make_async_copy(
                        rows_v.at[b2], out_hbm.at[pl.ds(off2, _CHUNK)],
                        ssem.at[b2]).wait()

                @pl.when(ck < nch - 2)
                def _():
                    start_gather(ck + 2, b2)

                off = base + ck * _CHUNK
                pltpu.make_async_copy(
                    table_hbm.at[idx_v.at[b]], rows_v.at[b],
                    gsem.at[b]).wait()
                pltpu.async_copy(rows_v.at[b], out_hbm.at[pl.ds(off, _CHUNK)],
                                 ssem.at[b])
            return 0

        lax.fori_loop(0, nch // 4, body, 0)

        for ck in (nch - 2, nch - 1):
            b = ck % 4
            off = base + ck * _CHUNK
            pltpu.make_async_copy(
                rows_v.at[b], out_hbm.at[pl.ds(off, _CHUNK)],
                ssem.at[b]).wait()

    return k


def _transpose_body(x_ref, o_ref):
    xt = jnp.transpose(x_ref[...])
    o_ref[...] = (jnp.concatenate([xt[:_EMB, :], xt[_EMB:, :]], axis=1)
                  * _SCALE).reshape(1, _EMB, 1024)


@functools.lru_cache(maxsize=None)
def _transpose_kernel():
    nblk = _BATCH // 1024
    return pl.pallas_call(
        _transpose_body,
        grid=(_SEQ, nblk),
        in_specs=[pl.BlockSpec((512, 128), lambda s, b: (s * nblk + b, 0))],
        out_specs=pl.BlockSpec((1, _EMB, 1024), lambda s, b: (s, 0, b)),
        out_shape=jax.ShapeDtypeStruct((_SEQ, _EMB, _BATCH), jnp.float32),
    )


@jax.jit
def kernel(tokens, table):
    tok = jnp.transpose(tokens, (1, 0)).reshape(-1).astype(jnp.int32)
    tmp = _gather_kernel()(tok, table)
    tmp2 = tmp.reshape(_B // 2, 128)
    x = _transpose_kernel()(tmp2)
    return jnp.transpose(x, (2, 0, 1))


# speedup vs baseline: 2.9142x; 2.0865x over previous
import functools

import jax
import jax.numpy as jnp
from jax import lax
from jax.experimental import pallas as pl
from jax.experimental.pallas import tpu as pltpu
from jax.experimental.pallas import tpu_sc as plsc

_EMB = 64
_SCALE = 8.0

_NC = 2
_NS = 16
_NW = _NC * _NS

_SEQ = 200
_BATCH = 16384
_B = _SEQ * _BATCH
_CHUNK = 256
_PITCH = 65


@functools.lru_cache(maxsize=None)
def _emb_kernel():
    bpw = _BATCH // _NW
    cps = bpw // _CHUNK
    nch = _SEQ * cps
    nq = _CHUNK // 128 * 8
    ncol = _CHUNK // 128
    mesh = plsc.VectorSubcoreMesh(core_axis_name="c", subcore_axis_name="s")

    @functools.partial(
        pl.kernel,
        mesh=mesh,
        compiler_params=pltpu.CompilerParams(use_tc_tiling_on_sc=False,
                                             needs_layout_passes=False),
        out_type=jax.ShapeDtypeStruct((_SEQ, 8, 1024, 128), jnp.float32),
        scratch_types=[
            pltpu.VMEM((4, _CHUNK), jnp.int32),
            pltpu.VMEM((4, _CHUNK, _EMB), jnp.float32),
            pltpu.VMEM((_CHUNK * _PITCH,), jnp.float32),
            pltpu.VMEM((2, 8, nq, 128), jnp.float32),
            pltpu.SemaphoreType.DMA((4,)),
            pltpu.SemaphoreType.DMA((2,)),
        ],
    )
    def k(tok_hbm, table_hbm, out_hbm, idx_v, rows_v, pad_v, t_v, gsem, wsem):
        wid = lax.axis_index("s") * _NC + lax.axis_index("c")

        def tok_off(ck):
            s = ck // cps
            c = ck % cps
            return s * _BATCH + wid * bpw + c * _CHUNK

        def start_gather(ck, b):
            off = pl.multiple_of(tok_off(ck), _CHUNK)
            pltpu.sync_copy(tok_hbm.at[pl.ds(off, _CHUNK)], idx_v.at[b])
            pltpu.async_copy(table_hbm.at[idx_v.at[b]], rows_v.at[b],
                             gsem.at[b])

        def out_slice(ck):
            s = ck // cps
            c = ck % cps
            q0 = wid * (bpw // 128 * 8) + c * nq
            return out_hbm.at[s, :, pl.ds(q0, nq), :]

        for b in range(3):
            start_gather(b, b)

        def body(g, _):
            for b in range(4):
                ck = g * 4 + b
                tb = b % 2

                @pl.when(ck < nch - 3)
                def _():
                    start_gather(ck + 3, (b + 3) % 4)

                pltpu.make_async_copy(
                    table_hbm.at[idx_v.at[b]], rows_v.at[b],
                    gsem.at[b]).wait()

                @plsc.parallel_loop(0, _CHUNK * (_EMB // 16), 1, unroll=8)
                def _(j):
                    t = j >> 2
                    c16 = (j & 3) * 16
                    pad_v[pl.ds(t * _PITCH + c16, 16)] = (
                        rows_v[b, t, pl.ds(c16, 16)])

                @pl.when(ck >= 2)
                def _():
                    pltpu.make_async_copy(
                        t_v.at[tb], out_slice(ck - 2), wsem.at[tb]).wait()

                @plsc.parallel_loop(0, _CHUNK * (_EMB // 16), 1, unroll=8)
                def _(j):
                    te = j >> 7
                    tcp = (j >> 6) & (ncol - 1)
                    ep = (j >> 3) & 7
                    rb = j & 7
                    ridx = ((tcp * 128 + rb * 16 + lax.iota(jnp.int32, 16))
                            * _PITCH + te * 8 + ep)
                    vals = plsc.load_gather(pad_v, [ridx])
                    t_v[tb, te, tcp * 8 + ep, pl.ds(rb * 16, 16)] = (
                        vals * _SCALE)

                pltpu.async_copy(t_v.at[tb], out_slice(ck), wsem.at[tb])
            return 0

        lax.fori_loop(0, nch // 4, body, 0)

        for ck in (nch - 2, nch - 1):
            pltpu.make_async_copy(
                t_v.at[ck % 2], out_slice(ck), wsem.at[ck % 2]).wait()

    return k


@jax.jit
def kernel(tokens, table):
    tok = jnp.transpose(tokens, (1, 0)).reshape(-1).astype(jnp.int32)
    x = _emb_kernel()(tok, table)
    x5 = x.reshape(_SEQ, 8, 128, 8, 128)
    return jnp.transpose(x5, (2, 4, 0, 1, 3)).reshape(_BATCH, _SEQ, _EMB)


# speedup vs baseline: 2.9639x; 1.0170x over previous
import functools

import jax
import jax.numpy as jnp
from jax import lax
from jax.experimental import pallas as pl
from jax.experimental.pallas import tpu as pltpu
from jax.experimental.pallas import tpu_sc as plsc

_EMB = 64
_SCALE = 8.0

_NC = 2
_NS = 16
_NW = _NC * _NS

_SEQ = 200
_BATCH = 16384
_B = _SEQ * _BATCH
_CHUNK = 256
_PITCH = 65


@functools.lru_cache(maxsize=None)
def _emb_kernel():
    bpw = _BATCH // _NW
    cps = bpw // _CHUNK
    nch = _SEQ * cps
    nq = _CHUNK // 128 * 8
    ncol = _CHUNK // 128
    mesh = plsc.VectorSubcoreMesh(core_axis_name="c", subcore_axis_name="s")

    @functools.partial(
        pl.kernel,
        mesh=mesh,
        compiler_params=pltpu.CompilerParams(use_tc_tiling_on_sc=False,
                                             needs_layout_passes=False),
        out_type=jax.ShapeDtypeStruct((_SEQ, 8, 1024, 128), jnp.float32),
        scratch_types=[
            pltpu.VMEM((4, _CHUNK), jnp.int32),
            pltpu.VMEM((4, _CHUNK, _EMB), jnp.float32),
            pltpu.VMEM((_CHUNK * _PITCH,), jnp.float32),
            pltpu.VMEM((2, 8, nq, 128), jnp.float32),
            pltpu.SemaphoreType.DMA((4,)),
            pltpu.SemaphoreType.DMA((2,)),
        ],
    )
    def k(tok_hbm, table_hbm, out_hbm, idx_v, rows_v, pad_v, t_v, gsem, wsem):
        wid = lax.axis_index("s") * _NC + lax.axis_index("c")

        def tok_off(ck):
            s = ck // cps
            c = ck % cps
            return s * _BATCH + wid * bpw + c * _CHUNK

        def start_gather(ck, b):
            off = pl.multiple_of(tok_off(ck), _CHUNK)
            pltpu.sync_copy(tok_hbm.at[pl.ds(off, _CHUNK)], idx_v.at[b])
            pltpu.async_copy(table_hbm.at[idx_v.at[b]], rows_v.at[b],
                             gsem.at[b])

        def out_slice(ck):
            s = ck // cps
            c = ck % cps
            q0 = wid * (bpw // 128 * 8) + c * nq
            return out_hbm.at[s, :, pl.ds(q0, nq), :]

        for b in range(3):
            start_gather(b, b)

        def body(g, _):
            for b in range(4):
                ck = g * 4 + b
                tb = b % 2

                @pl.when(ck < nch - 3)
                def _():
                    start_gather(ck + 3, (b + 3) % 4)

                pltpu.make_async_copy(
                    table_hbm.at[idx_v.at[b]], rows_v.at[b],
                    gsem.at[b]).wait()

                @plsc.parallel_loop(0, _CHUNK * (_EMB // 16), 1, unroll=16)
                def _(j):
                    t = j >> 2
                    c16 = (j & 3) * 16
                    pad_v[pl.ds(t * _PITCH + c16, 16)] = (
                        rows_v[b, t, pl.ds(c16, 16)])

                @pl.when(ck >= 2)
                def _():
                    pltpu.make_async_copy(
                        t_v.at[tb], out_slice(ck - 2), wsem.at[tb]).wait()

                @plsc.parallel_loop(0, _CHUNK * (_EMB // 16), 1, unroll=16)
                def _(j):
                    te = j >> 7
                    tcp = (j >> 6) & (ncol - 1)
                    ep = (j >> 3) & 7
                    rb = j & 7
                    ridx = ((tcp * 128 + rb * 16 + lax.iota(jnp.int32, 16))
                            * _PITCH + te * 8 + ep)
                    vals = plsc.load_gather(pad_v, [ridx])
                    t_v[tb, te, tcp * 8 + ep, pl.ds(rb * 16, 16)] = (
                        vals * _SCALE)

                pltpu.async_copy(t_v.at[tb], out_slice(ck), wsem.at[tb])
            return 0

        lax.fori_loop(0, nch // 4, body, 0)

        for ck in (nch - 2, nch - 1):
            pltpu.make_async_copy(
                t_v.at[ck % 2], out_slice(ck), wsem.at[ck % 2]).wait()

    return k


@jax.jit
def kernel(tokens, table):
    tok = jnp.transpose(tokens, (1, 0)).reshape(-1).astype(jnp.int32)
    x = _emb_kernel()(tok, table)
    x5 = x.reshape(_SEQ, 8, 128, 8, 128)
    return jnp.transpose(x5, (2, 4, 0, 1, 3)).reshape(_BATCH, _SEQ, _EMB)


# speedup vs baseline: 3.0883x; 1.0420x over previous
import functools

import jax
import jax.numpy as jnp
from jax import lax
from jax.experimental import pallas as pl
from jax.experimental.pallas import tpu as pltpu
from jax.experimental.pallas import tpu_sc as plsc

_EMB = 64
_SCALE = 8.0

_NC = 2
_NS = 16
_NW = _NC * _NS

_SEQ = 200
_BATCH = 16384
_B = _SEQ * _BATCH
_CHUNK = 256
_PITCH = 65


@functools.lru_cache(maxsize=None)
def _emb_kernel():
    bpw = _BATCH // _NW
    cps = bpw // _CHUNK
    nch = _SEQ * cps
    nq = _CHUNK // 128 * 8
    ncol = _CHUNK // 128
    mesh = plsc.VectorSubcoreMesh(core_axis_name="c", subcore_axis_name="s")

    @functools.partial(
        pl.kernel,
        mesh=mesh,
        compiler_params=pltpu.CompilerParams(use_tc_tiling_on_sc=False,
                                             needs_layout_passes=False),
        out_type=jax.ShapeDtypeStruct((_SEQ, 8, 1024, 128), jnp.float32),
        scratch_types=[
            pltpu.VMEM((4, _CHUNK), jnp.int32),
            pltpu.VMEM((4, _CHUNK, _EMB), jnp.float32),
            pltpu.VMEM((_CHUNK * _PITCH,), jnp.float32),
            pltpu.VMEM((2, 8, nq, 128), jnp.float32),
            pltpu.SemaphoreType.DMA((4,)),
            pltpu.SemaphoreType.DMA((2,)),
        ],
    )
    def k(tok_hbm, table_hbm, out_hbm, idx_v, rows_v, pad_v, t_v, gsem, wsem):
        wid = lax.axis_index("s") * _NC + lax.axis_index("c")

        def tok_off(ck):
            s = ck // cps
            c = ck % cps
            return s * _BATCH + wid * bpw + c * _CHUNK

        def start_gather(ck, b):
            off = pl.multiple_of(tok_off(ck), _CHUNK)
            pltpu.sync_copy(tok_hbm.at[pl.ds(off, _CHUNK)], idx_v.at[b])
            pltpu.async_copy(table_hbm.at[idx_v.at[b]], rows_v.at[b],
                             gsem.at[b])

        def out_slice(ck):
            s = ck // cps
            c = ck % cps
            q0 = wid * (bpw // 128 * 8) + c * nq
            return out_hbm.at[s, :, pl.ds(q0, nq), :]

        for b in range(3):
            start_gather(b, b)

        def body(g, _):
            for b in range(4):
                ck = g * 4 + b
                tb = b % 2

                @pl.when(ck < nch - 3)
                def _():
                    start_gather(ck + 3, (b + 3) % 4)

                pltpu.make_async_copy(
                    table_hbm.at[idx_v.at[b]], rows_v.at[b],
                    gsem.at[b]).wait()

                @plsc.parallel_loop(0, _CHUNK * (_EMB // 16), 1, unroll=32)
                def _(j):
                    t = j >> 2
                    c16 = (j & 3) * 16
                    pad_v[pl.ds(t * _PITCH + c16, 16)] = (
                        rows_v[b, t, pl.ds(c16, 16)])

                @pl.when(ck >= 2)
                def _():
                    pltpu.make_async_copy(
                        t_v.at[tb], out_slice(ck - 2), wsem.at[tb]).wait()

                @plsc.parallel_loop(0, _CHUNK * (_EMB // 16), 1, unroll=32)
                def _(j):
                    te = j >> 7
                    tcp = (j >> 6) & (ncol - 1)
                    ep = (j >> 3) & 7
                    rb = j & 7
                    ridx = ((tcp * 128 + rb * 16 + lax.iota(jnp.int32, 16))
                            * _PITCH + te * 8 + ep)
                    vals = plsc.load_gather(pad_v, [ridx])
                    t_v[tb, te, tcp * 8 + ep, pl.ds(rb * 16, 16)] = (
                        vals * _SCALE)

                pltpu.async_copy(t_v.at[tb], out_slice(ck), wsem.at[tb])
            return 0

        lax.fori_loop(0, nch // 4, body, 0)

        for ck in (nch - 2, nch - 1):
            pltpu.make_async_copy(
                t_v.at[ck % 2], out_slice(ck), wsem.at[ck % 2]).wait()

    return k


@jax.jit
def kernel(tokens, table):
    tok = jnp.transpose(tokens, (1, 0)).reshape(-1).astype(jnp.int32)
    x = _emb_kernel()(tok, table)
    x5 = x.reshape(_SEQ, 8, 128, 8, 128)
    return jnp.transpose(x5, (2, 4, 0, 1, 3)).reshape(_BATCH, _SEQ, _EMB)


# speedup vs baseline: 3.3632x; 1.0890x over previous
import functools

import jax
import jax.numpy as jnp
from jax import lax
from jax.experimental import pallas as pl
from jax.experimental.pallas import tpu as pltpu
from jax.experimental.pallas import tpu_sc as plsc

_EMB = 64
_SCALE = 8.0

_NC = 2
_NS = 16
_NW = _NC * _NS

_SEQ = 200
_BATCH = 16384
_B = _SEQ * _BATCH
_CHUNK = 256
_PITCH = 65


@functools.lru_cache(maxsize=None)
def _emb_kernel():
    bpw = _BATCH // _NW
    cps = bpw // _CHUNK
    nch = _SEQ * cps
    nq = _CHUNK // 128 * 8
    ncol = _CHUNK // 128
    mesh = plsc.VectorSubcoreMesh(core_axis_name="c", subcore_axis_name="s")

    @functools.partial(
        pl.kernel,
        mesh=mesh,
        compiler_params=pltpu.CompilerParams(use_tc_tiling_on_sc=False,
                                             needs_layout_passes=False),
        out_type=jax.ShapeDtypeStruct((_SEQ, 8, 1024, 128), jnp.float32),
        scratch_types=[
            pltpu.VMEM((4, _CHUNK), jnp.int32),
            pltpu.VMEM((4, _CHUNK, _EMB), jnp.float32),
            pltpu.VMEM((_CHUNK * _PITCH,), jnp.float32),
            pltpu.VMEM((2, 8, nq, 128), jnp.float32),
            pltpu.SemaphoreType.DMA((4,)),
            pltpu.SemaphoreType.DMA((2,)),
        ],
    )
    def k(tok_hbm, table_hbm, out_hbm, idx_v, rows_v, pad_v, t_v, gsem, wsem):
        wid = lax.axis_index("s") * _NC + lax.axis_index("c")

        def tok_off(ck):
            s = ck // cps
            c = ck % cps
            return s * _BATCH + wid * bpw + c * _CHUNK

        def start_gather(ck, b):
            off = pl.multiple_of(tok_off(ck), _CHUNK)
            pltpu.sync_copy(tok_hbm.at[pl.ds(off, _CHUNK)], idx_v.at[b])
            pltpu.async_copy(table_hbm.at[idx_v.at[b]], rows_v.at[b],
                             gsem.at[b])

        def out_slice(ck):
            s = ck // cps
            c = ck % cps
            q0 = wid * (bpw // 128 * 8) + c * nq
            return out_hbm.at[s, :, pl.ds(q0, nq), :]

        for b in range(3):
            start_gather(b, b)

        def body(g, _):
            for b in range(4):
                ck = g * 4 + b
                tb = b % 2

                @pl.when(ck < nch - 3)
                def _():
                    start_gather(ck + 3, (b + 3) % 4)

                pltpu.make_async_copy(
                    table_hbm.at[idx_v.at[b]], rows_v.at[b],
                    gsem.at[b]).wait()

                @plsc.parallel_loop(0, _CHUNK * (_EMB // 16), 1, unroll=64)
                def _(j):
                    t = j >> 2
                    c16 = (j & 3) * 16
                    pad_v[pl.ds(t * _PITCH + c16, 16)] = (
                        rows_v[b, t, pl.ds(c16, 16)])

                @pl.when(ck >= 2)
                def _():
                    pltpu.make_async_copy(
                        t_v.at[tb], out_slice(ck - 2), wsem.at[tb]).wait()

                @plsc.parallel_loop(0, _CHUNK * (_EMB // 16), 1, unroll=64)
                def _(j):
                    te = j >> 7
                    tcp = (j >> 6) & (ncol - 1)
                    ep = (j >> 3) & 7
                    rb = j & 7
                    ridx = ((tcp * 128 + rb * 16 + lax.iota(jnp.int32, 16))
                            * _PITCH + te * 8 + ep)
                    vals = plsc.load_gather(pad_v, [ridx])
                    t_v[tb, te, tcp * 8 + ep, pl.ds(rb * 16, 16)] = (
                        vals * _SCALE)

                pltpu.async_copy(t_v.at[tb], out_slice(ck), wsem.at[tb])
            return 0

        lax.fori_loop(0, nch // 4, body, 0)

        for ck in (nch - 2, nch - 1):
            pltpu.make_async_copy(
                t_v.at[ck % 2], out_slice(ck), wsem.at[ck % 2]).wait()

    return k


@jax.jit
def kernel(tokens, table):
    tok = jnp.transpose(tokens, (1, 0)).reshape(-1).astype(jnp.int32)
    x = _emb_kernel()(tok, table)
    x5 = x.reshape(_SEQ, 8, 128, 8, 128)
    return jnp.transpose(x5, (2, 4, 0, 1, 3)).reshape(_BATCH, _SEQ, _EMB)
